# Initial kernel scaffold; baseline (speedup 1.0000x reference)
import jax
import jax.numpy as jnp
from jax.experimental import pallas as pl


def kernel(probs, action):
    raise NotImplementedError("write your pallas kernel here")


# speedup vs baseline: 1.7229x; 1.7229x over previous
import functools

import jax
import jax.numpy as jnp
from jax.experimental import pallas as pl
from jax.experimental.pallas import tpu as pltpu

_CC = 27776


def _lse_update(x, m, s):
    bm = jnp.max(x, axis=1, keepdims=True)
    m2 = jnp.maximum(m, bm)
    s2 = s * jnp.exp(m - m2) + jnp.sum(jnp.exp(x - m2), axis=1, keepdims=True)
    return m2, s2


def _logpf(p, ac):
    pe = p + jnp.float32(1e-20)
    return ac * jnp.log1p(-pe) + jnp.log(pe)


def _body(action_ref, probs_ref, logpf_ref, logpb_ref,
          b0, b1, f0, f1, o0, o1, strad, otail, acol,
          sb0, sb1, sf0, sf1, so0, so1, sst, sot, sac,
          *, B, L, CC):
    body = L - 64
    bback = L + 64
    nchunk = body // CC
    a = action_ref[0]
    ac = a.astype(jnp.float32)

    def start_in(c, bbuf, fbuf, sb, sf):
        pltpu.make_async_copy(
            probs_ref.at[:, pl.ds(bback + c * CC, CC)], bbuf, sb).start()
        pltpu.make_async_copy(
            probs_ref.at[pl.ds(0, 1), pl.ds(c * CC, CC)], fbuf, sf).start()

    pltpu.make_async_copy(
        probs_ref.at[:, pl.ds(body, 128)], strad, sst).start()
    astart = pl.multiple_of(((L + a) // 128) * 128, 128)
    pltpu.make_async_copy(
        probs_ref.at[:, pl.ds(astart, 128)], acol, sac).start()
    start_in(0, b0, f0, sb0, sf0)
    start_in(1, b1, f1, sb1, sf1)

    pltpu.make_async_copy(
        probs_ref.at[:, pl.ds(body, 128)], strad, sst).wait()
    lane = jax.lax.broadcasted_iota(jnp.int32, (B, 128), 1)
    xh = jnp.where(lane >= 64, strad[...], -jnp.inf)
    m0 = jnp.max(xh, axis=1, keepdims=True)
    s0 = jnp.sum(jnp.where(lane >= 64, jnp.exp(strad[...] - m0), 0.0),
                 axis=1, keepdims=True)
    otail[...] = _logpf(strad[pl.ds(0, 1), pl.ds(0, 64)], ac)
    pltpu.make_async_copy(
        otail, logpf_ref.at[:, pl.ds(body, 64)], sot).start()

    def step(c, bbuf, fbuf, obuf, sb, sf, so, m, s, first):
        pltpu.make_async_copy(
            probs_ref.at[:, pl.ds(bback + c * CC, CC)], bbuf, sb).wait()
        pltpu.make_async_copy(
            probs_ref.at[pl.ds(0, 1), pl.ds(c * CC, CC)], fbuf, sf).wait()
        m, s = _lse_update(bbuf[...], m, s)

        @pl.when(jnp.logical_not(first))
        def _():
            pltpu.make_async_copy(
                obuf, logpf_ref.at[:, pl.ds((c - 2) * CC, CC)], so).wait()
        obuf[...] = _logpf(fbuf[...], ac)
        pltpu.make_async_copy(
            obuf, logpf_ref.at[:, pl.ds(c * CC, CC)], so).start()

        @pl.when(c + 2 < nchunk)
        def _():
            start_in(c + 2, bbuf, fbuf, sb, sf)
        return m, s

    def loop(i2, carry):
        m, s = carry
        c0 = 2 * i2
        m, s = step(c0, b0, f0, o0, sb0, sf0, so0, m, s, i2 == 0)
        m, s = step(c0 + 1, b1, f1, o1, sb1, sf1, so1, m, s, i2 == 0)
        return m, s

    m, s = jax.lax.fori_loop(0, nchunk // 2, loop, (m0, s0))

    pltpu.make_async_copy(
        o0, logpf_ref.at[:, pl.ds((nchunk - 2) * CC, CC)], so0).wait()
    pltpu.make_async_copy(
        o1, logpf_ref.at[:, pl.ds((nchunk - 1) * CC, CC)], so1).wait()
    pltpu.make_async_copy(
        otail, logpf_ref.at[:, pl.ds(body, 64)], sot).wait()
    pltpu.make_async_copy(
        probs_ref.at[:, pl.ds(astart, 128)], acol, sac).wait()

    off = (L + a) - astart
    bval = jnp.sum(jnp.where(lane == off, acol[...], 0.0), axis=1,
                   keepdims=True)
    logpb_ref[...] = bval - (m + jnp.log(s))


def kernel(probs, action):
    B, twoL = probs.shape
    L = twoL // 2
    CC = _CC
    assert L % 128 == 64 and (L - 64) % CC == 0 and ((L - 64) // CC) % 2 == 0
    a = jnp.asarray(action, jnp.int32).reshape(1)
    logpf, logpb = pl.pallas_call(
        functools.partial(_body, B=B, L=L, CC=CC),
        in_specs=[
            pl.BlockSpec(memory_space=pltpu.SMEM),
            pl.BlockSpec(memory_space=pl.ANY),
        ],
        out_specs=[
            pl.BlockSpec(memory_space=pl.ANY),
            pl.BlockSpec(memory_space=pltpu.VMEM),
        ],
        out_shape=[
            jax.ShapeDtypeStruct((1, L), jnp.float32),
            jax.ShapeDtypeStruct((B, 1), jnp.float32),
        ],
        scratch_shapes=(
            [pltpu.VMEM((B, CC), jnp.float32)] * 2
            + [pltpu.VMEM((1, CC), jnp.float32)] * 4
            + [pltpu.VMEM((B, 128), jnp.float32)]
            + [pltpu.VMEM((1, 64), jnp.float32)]
            + [pltpu.VMEM((B, 128), jnp.float32)]
            + [pltpu.SemaphoreType.DMA] * 9
        ),
    )(a, probs)
    return logpf.reshape(L), logpb.reshape(B)
